# baseline (speedup 1.0000x reference)
import functools

import jax
import jax.numpy as jnp
from jax import lax
from jax.experimental import pallas as pl
from jax.experimental.pallas import tpu as pltpu
from jax.experimental.pallas import tpu_sc as plsc

F = 128


def _stage1_body(q_ref, mu_ref, W1_ref, b1_ref, W2_ref, b2_ref,
                 Wc1_ref, bc1_ref, Wc2_ref, bc2_ref,
                 x_ref, w_ref, s_ref):
    q2 = q_ref[:, 0, :]
    a = jnp.dot(q2, W1_ref[...], preferred_element_type=jnp.float32) + b1_ref[...]
    a = a * jax.nn.sigmoid(a)
    x = jnp.dot(a, W2_ref[...], preferred_element_type=jnp.float32) + b2_ref[...]
    mu = mu_ref[...]
    inv = jnp.sqrt(mu[:, 0, :] ** 2 + mu[:, 1, :] ** 2 + mu[:, 2, :] ** 2)
    h = jnp.maximum(jnp.dot(inv, Wc1_ref[...], preferred_element_type=jnp.float32)
                    + bc1_ref[...], 0.0)
    logit = jnp.dot(h, Wc2_ref[...], preferred_element_type=jnp.float32) + bc2_ref[...]
    m = jnp.max(logit, axis=-1, keepdims=True)
    e = jnp.exp(logit - m)
    w = e / jnp.sum(e, axis=-1, keepdims=True)
    w_ref[...] = w
    x_ref[...] = jnp.concatenate(
        [x[:, 0:256], x[:, 256:384] * w], axis=-1)
    s_ref[...] = jnp.sum(mu, axis=2)


def _stage1(q, mu, W1, b1, W2, b2, Wc1, bc1, Wc2, bc2):
    N = q.shape[0]
    B = 400
    grid = N // B
    full = lambda *shape: pl.BlockSpec(shape, lambda i: (0,) * len(shape))
    return pl.pallas_call(
        _stage1_body,
        grid=(grid,),
        in_specs=[
            pl.BlockSpec((B, 1, F), lambda i: (i, 0, 0)),
            pl.BlockSpec((B, 3, F), lambda i: (i, 0, 0)),
            full(F, F), full(1, F), full(F, 3 * F), full(1, 3 * F),
            full(F, 32), full(1, 32), full(32, F), full(1, F),
        ],
        out_specs=[
            pl.BlockSpec((B, 3 * F), lambda i: (i, 0)),
            pl.BlockSpec((B, F), lambda i: (i, 0)),
            pl.BlockSpec((B, 3), lambda i: (i, 0)),
        ],
        out_shape=[
            jax.ShapeDtypeStruct((N, 3 * F), jnp.float32),
            jax.ShapeDtypeStruct((N, F), jnp.float32),
            jax.ShapeDtypeStruct((N, 3), jnp.float32),
        ],
    )(q, mu, W1, b1.reshape(1, -1), W2, b2.reshape(1, -1),
      Wc1, bc1.reshape(1, -1), Wc2, bc2.reshape(1, -1))


def _sc_edges(table, idxj4, idx_i, wij, dir_ij, zeros):
    E = idx_i.shape[0]
    N = table.shape[0] // 4
    K = 80
    NB = E // K
    assert NB * K == E and K % 8 == 0 and NB % 2 == 0
    NW = 16
    NBH = NB // 2
    total_k = (NBH + NW - 1) // NW
    total_k += total_k % 2
    rpt = N // 16
    mesh = plsc.VectorSubcoreMesh(core_axis_name="c", subcore_axis_name="s",
                                  num_cores=1)

    slot_t = [
        pltpu.VMEM((K, 112), jnp.float32),
        pltpu.VMEM((K, 96), jnp.float32),
        pltpu.VMEM((K, 16), jnp.float32),
        pltpu.VMEM((K,), jnp.int32),
        pltpu.VMEM((K,), jnp.int32),
        pltpu.SemaphoreType.DMA,
    ]

    def make_half(half):
      @functools.partial(
        pl.kernel, mesh=mesh,
        compiler_params=pltpu.CompilerParams(use_tc_tiling_on_sc=False),
        out_type=jax.ShapeDtypeStruct((4, N, F), jnp.float32),
        scratch_types=slot_t + slot_t + [
            pltpu.VMEM((K, F), jnp.float32),
            pltpu.VMEM_SHARED((N, F), jnp.float32),
        ],
      )
      def sc_fn(table_hbm, idxj_hbm, idxi_hbm, wij_hbm, dir_hbm, zeros_hbm,
                out_hbm,
                rows_a, wij_a, dir_a, idxj_a, idxi_a, sem_a,
                rows_b, wij_b, dir_b, idxj_b, idxi_b, sem_b,
                contrib_v, acc):
        wid = lax.axis_index("s")
        slots = ((rows_a, wij_a, dir_a, idxj_a, idxi_a, sem_a),
                 (rows_b, wij_b, dir_b, idxj_b, idxi_b, sem_b))

        def issue(p, t, slot):
            rows_v, wij_v, dir_v, idxj_v, idxi_v, sem = slot
            base = t * K
            pltpu.sync_copy(idxj_hbm.at[pl.ds(p * E + base, K)], idxj_v)
            pltpu.async_copy(table_hbm.at[idxj_v], rows_v, sem)
            pltpu.async_copy(idxi_hbm.at[pl.ds(base, K)], idxi_v, sem)
            pltpu.async_copy(wij_hbm.at[pl.ds(base, K), pl.ds(p * 32, 32)],
                             wij_v.at[:, pl.ds(0, 32)], sem)
            pltpu.async_copy(wij_hbm.at[pl.ds(base, K), pl.ds(128 + p * 32, 32)],
                             wij_v.at[:, pl.ds(32, 32)], sem)
            pltpu.async_copy(wij_hbm.at[pl.ds(base, K), pl.ds(256 + p * 32, 32)],
                             wij_v.at[:, pl.ds(64, 32)], sem)
            pltpu.async_copy(dir_hbm.at[pl.ds(base, K)], dir_v, sem)

        def process(slot):
            rows_v, wij_v, dir_v, idxj_v, idxi_v, sem = slot
            pltpu.make_async_copy(table_hbm.at[idxj_v], rows_v, sem).wait()
            pltpu.make_async_copy(idxi_hbm.at[pl.ds(0, K)], idxi_v, sem).wait()
            pltpu.make_async_copy(wij_hbm.at[pl.ds(0, K), pl.ds(0, 96)],
                                  wij_v, sem).wait()
            pltpu.make_async_copy(dir_hbm.at[pl.ds(0, K)], dir_v, sem).wait()

            def edge_body(e, _):
                sv = rows_v[e, pl.ds(96, 16)]
                s0 = sv[0]
                s1 = sv[1]
                s2 = sv[2]
                dv = dir_v[e, pl.ds(0, 16)]
                d0 = dv[0]
                d1 = dv[1]
                d2 = dv[2]
                for h in (0, 16):
                    x1 = rows_v[e, pl.ds(h, 16)]
                    x2 = rows_v[e, pl.ds(32 + h, 16)]
                    x3 = rows_v[e, pl.ds(64 + h, 16)]
                    wq = wij_v[e, pl.ds(h, 16)]
                    wr = wij_v[e, pl.ds(32 + h, 16)]
                    wm = wij_v[e, pl.ds(64 + h, 16)]
                    av = wr * x2
                    bv = wm * x3
                    contrib_v[e, pl.ds(h, 16)] = wq * x1
                    contrib_v[e, pl.ds(32 + h, 16)] = av * d0 + bv * s0
                    contrib_v[e, pl.ds(64 + h, 16)] = av * d1 + bv * s1
                    contrib_v[e, pl.ds(96 + h, 16)] = av * d2 + bv * s2
                return 0

            lax.fori_loop(0, K, edge_body, 0)
            pltpu.sync_copy(contrib_v, acc.at[idxi_v], add=True)

        t0 = half * NBH
        t1 = (half + 1) * NBH
        for p in range(4):
            pltpu.sync_copy(zeros_hbm, acc.at[pl.ds(wid * rpt, rpt)])
            plsc.subcore_barrier()
            issue(p, t0 + wid, slots[0])

            def pair_body(i, _, p=p):
                t_a = t0 + wid + (2 * i) * NW
                t_b = t0 + wid + (2 * i + 1) * NW
                t_a2 = t0 + wid + (2 * i + 2) * NW

                @pl.when(t_b < t1)
                def _():
                    issue(p, t_b, slots[1])

                @pl.when(t_a < t1)
                def _():
                    process(slots[0])

                @pl.when(t_a2 < t1)
                def _():
                    issue(p, t_a2, slots[0])

                @pl.when(t_b < t1)
                def _():
                    process(slots[1])

                return 0

            lax.fori_loop(0, total_k // 2, pair_body, 0)
            plsc.subcore_barrier()
            pltpu.sync_copy(acc.at[pl.ds(wid * rpt, rpt)],
                            out_hbm.at[p, pl.ds(wid * rpt, rpt)])
            plsc.subcore_barrier()

      return sc_fn

    h0 = make_half(0)(table, idxj4, idx_i, wij, dir_ij, zeros)
    h1 = make_half(1)(table, idxj4, idx_i, wij, dir_ij, zeros)
    return jnp.stack([h0, h1], axis=0)


def _stage3_body(q_ref, dq_ref, dmu_ref, w_ref, s_ref,
                 Wr1_ref, br1_ref, Wr2T_ref, br2T_ref,
                 qn_ref, mun_ref, dtm_ref):
    B = q_ref.shape[0]
    w = w_ref[...]
    s = s_ref[...]
    mu_new = dmu_ref[...] + w[:, None, :] * s[:, :, None]
    mun_ref[...] = mu_new
    qn_ref[...] = q_ref[...] + dq_ref[...][:, None, :]
    inv2 = jnp.sqrt(mu_new[:, 0, :] ** 2 + mu_new[:, 1, :] ** 2
                    + mu_new[:, 2, :] ** 2)
    h2 = jnp.maximum(jnp.dot(inv2, Wr1_ref[...], preferred_element_type=jnp.float32)
                     + br1_ref[...], 0.0)
    sw2t = jnp.dot(h2, Wr2T_ref[...], preferred_element_type=jnp.float32) \
        + br2T_ref[...]
    zrows = jnp.zeros((7, F), jnp.float32)
    for b in range(B):
        mb0 = mu_new[b:b + 1, 0, :]
        mb1 = mu_new[b:b + 1, 1, :]
        mb2 = mu_new[b:b + 1, 2, :]
        rows = [mb0 * mb0, mb0 * mb1, mb0 * mb2,
                mb1 * mb0, mb1 * mb1, mb1 * mb2,
                mb2 * mb0, mb2 * mb1, mb2 * mb2]
        rt16 = jnp.concatenate(rows + [zrows], axis=0)
        swbt = sw2t[b:b + 1, :].reshape(F, F)
        dt16 = jnp.dot(rt16, swbt, preferred_element_type=jnp.float32)
        dtm_ref[b, :, :] = dt16[0:9, :]


def _stage3(q, dq, dmu, w, s, Wr1, br1, Wr2T, br2T):
    N = q.shape[0]
    B = 8
    grid = N // B
    full = lambda *shape: pl.BlockSpec(shape, lambda i: (0,) * len(shape))
    return pl.pallas_call(
        functools.partial(_stage3_body),
        grid=(grid,),
        in_specs=[
            pl.BlockSpec((B, 1, F), lambda i: (i, 0, 0)),
            pl.BlockSpec((B, F), lambda i: (i, 0)),
            pl.BlockSpec((B, 3, F), lambda i: (i, 0, 0)),
            pl.BlockSpec((B, F), lambda i: (i, 0)),
            pl.BlockSpec((B, 3), lambda i: (i, 0)),
            full(F, 32), full(1, 32), full(32, F * F), full(1, F * F),
        ],
        out_specs=[
            pl.BlockSpec((B, 1, F), lambda i: (i, 0, 0)),
            pl.BlockSpec((B, 3, F), lambda i: (i, 0, 0)),
            pl.BlockSpec((B, 9, F), lambda i: (i, 0, 0)),
        ],
        out_shape=[
            jax.ShapeDtypeStruct((N, 1, F), jnp.float32),
            jax.ShapeDtypeStruct((N, 3, F), jnp.float32),
            jax.ShapeDtypeStruct((N, 9, F), jnp.float32),
        ],
    )(q, dq, dmu, w, s, Wr1, br1.reshape(1, -1), Wr2T, br2T.reshape(1, -1))


def kernel(q, mu, Wij, dir_ij, idx_i, idx_j, n_atoms,
           W1, b1, W2, b2, Wc1, bc1, Wc2, bc2, Wr1, br1, Wr2, br2):
    N = q.shape[0]
    E = idx_i.shape[0]

    x, w, s = _stage1(q, mu, W1, b1, W2, b2, Wc1, bc1, Wc2, bc2)

    xr = x.reshape(N, 3, 4, 32).transpose(2, 0, 1, 3).reshape(4, N, 96)
    sr = jnp.broadcast_to(s[None], (4, N, 3))
    pad = jnp.zeros((4, N, 13), jnp.float32)
    table = jnp.concatenate([xr, sr, pad], axis=-1).reshape(4 * N, 112)
    idxj4 = (idx_j[None, :] + (jnp.arange(4, dtype=jnp.int32) * N)[:, None])
    idxj4 = idxj4.reshape(-1)
    zeros = jnp.zeros((N // 16, F), jnp.float32)
    dir16 = jnp.concatenate(
        [dir_ij, jnp.zeros((E, 13), jnp.float32)], axis=-1)

    partials = _sc_edges(table, idxj4, idx_i, Wij.reshape(E, 3 * F),
                         dir16, zeros)

    P = partials[0] + partials[1]
    Pr = P.reshape(4, N, 4, 32)
    dq = Pr[:, :, 0, :].transpose(1, 0, 2).reshape(N, F)
    dmu = Pr[:, :, 1:4, :].transpose(1, 2, 0, 3).reshape(N, 3, F)

    Wr2T = Wr2.reshape(32, F, F).transpose(0, 2, 1).reshape(32, F * F)
    br2T = br2.reshape(F, F).T.reshape(F * F)
    q_new, mu_new, dtm9 = _stage3(q, dq, dmu, w, s, Wr1, br1, Wr2T, br2T)
    dtm = dtm9.transpose(0, 2, 1).reshape(N, F, 3, 3)
    return (q_new, mu_new, dtm)


# speedup vs baseline: 1.4755x; 1.4755x over previous
import functools

import jax
import jax.numpy as jnp
from jax import lax
from jax.experimental import pallas as pl
from jax.experimental.pallas import tpu as pltpu
from jax.experimental.pallas import tpu_sc as plsc

F = 128


def _stage1_body(q_ref, mu_ref, W1_ref, b1_ref, W2_ref, b2_ref,
                 Wc1_ref, bc1_ref, Wc2_ref, bc2_ref,
                 x_ref, w_ref, s_ref):
    q2 = q_ref[:, 0, :]
    a = jnp.dot(q2, W1_ref[...], preferred_element_type=jnp.float32) + b1_ref[...]
    a = a * jax.nn.sigmoid(a)
    x = jnp.dot(a, W2_ref[...], preferred_element_type=jnp.float32) + b2_ref[...]
    mu = mu_ref[...]
    inv = jnp.sqrt(mu[:, 0, :] ** 2 + mu[:, 1, :] ** 2 + mu[:, 2, :] ** 2)
    h = jnp.maximum(jnp.dot(inv, Wc1_ref[...], preferred_element_type=jnp.float32)
                    + bc1_ref[...], 0.0)
    logit = jnp.dot(h, Wc2_ref[...], preferred_element_type=jnp.float32) + bc2_ref[...]
    m = jnp.max(logit, axis=-1, keepdims=True)
    e = jnp.exp(logit - m)
    w = e / jnp.sum(e, axis=-1, keepdims=True)
    w_ref[...] = w
    x_ref[...] = jnp.concatenate(
        [x[:, 0:256], x[:, 256:384] * w], axis=-1)
    s_ref[...] = jnp.sum(mu, axis=2)


def _stage1(q, mu, W1, b1, W2, b2, Wc1, bc1, Wc2, bc2):
    N = q.shape[0]
    B = 400
    grid = N // B
    full = lambda *shape: pl.BlockSpec(shape, lambda i: (0,) * len(shape))
    return pl.pallas_call(
        _stage1_body,
        grid=(grid,),
        in_specs=[
            pl.BlockSpec((B, 1, F), lambda i: (i, 0, 0)),
            pl.BlockSpec((B, 3, F), lambda i: (i, 0, 0)),
            full(F, F), full(1, F), full(F, 3 * F), full(1, 3 * F),
            full(F, 32), full(1, 32), full(32, F), full(1, F),
        ],
        out_specs=[
            pl.BlockSpec((B, 3 * F), lambda i: (i, 0)),
            pl.BlockSpec((B, F), lambda i: (i, 0)),
            pl.BlockSpec((B, 3), lambda i: (i, 0)),
        ],
        out_shape=[
            jax.ShapeDtypeStruct((N, 3 * F), jnp.float32),
            jax.ShapeDtypeStruct((N, F), jnp.float32),
            jax.ShapeDtypeStruct((N, 3), jnp.float32),
        ],
    )(q, mu, W1, b1.reshape(1, -1), W2, b2.reshape(1, -1),
      Wc1, bc1.reshape(1, -1), Wc2, bc2.reshape(1, -1))


def _sc_edges(table, idxj4, idx_i, wij, dir_ij, zeros):
    E = idx_i.shape[0]
    N = table.shape[0] // 4
    K = 64
    NB = E // K
    assert NB * K == E and K % 8 == 0
    NW = 32
    total_k = (NB + NW - 1) // NW
    total_k += total_k % 2
    rpt = N // 16
    mesh = plsc.VectorSubcoreMesh(core_axis_name="c", subcore_axis_name="s")

    slot_t = [
        pltpu.VMEM((K, 112), jnp.float32),
        pltpu.VMEM((K, 96), jnp.float32),
        pltpu.VMEM((K, 16), jnp.float32),
        pltpu.VMEM((K,), jnp.int32),
        pltpu.VMEM((K,), jnp.int32),
        pltpu.VMEM((K, F), jnp.float32),
        pltpu.SemaphoreType.DMA,
        pltpu.SemaphoreType.DMA,
    ]

    @functools.partial(
        pl.kernel, mesh=mesh,
        compiler_params=pltpu.CompilerParams(use_tc_tiling_on_sc=False),
        out_type=jax.ShapeDtypeStruct((2, 4, N, F), jnp.float32),
        scratch_types=slot_t + slot_t + [
            pltpu.VMEM_SHARED((N, F), jnp.float32),
        ],
    )
    def sc_fn(table_hbm, idxj_hbm, idxi_hbm, wij_hbm, dir_hbm, zeros_hbm,
              out_hbm,
              rows_a, wij_a, dir_a, idxj_a, idxi_a, ctb_a, sem_a, ssem_a,
              rows_b, wij_b, dir_b, idxj_b, idxi_b, ctb_b, sem_b, ssem_b,
              acc):
        cid = lax.axis_index("c")
        sid = lax.axis_index("s")
        wid = sid * 2 + cid
        slots = ((rows_a, wij_a, dir_a, idxj_a, idxi_a, ctb_a, sem_a, ssem_a),
                 (rows_b, wij_b, dir_b, idxj_b, idxi_b, ctb_b, sem_b, ssem_b))

        def issue(p, t, slot):
            rows_v, wij_v, dir_v, idxj_v, idxi_v, ctb_v, sem, ssem = slot
            base = t * K
            pltpu.sync_copy(idxj_hbm.at[pl.ds(p * E + base, K)], idxj_v)
            pltpu.async_copy(table_hbm.at[idxj_v], rows_v, sem)
            pltpu.async_copy(wij_hbm.at[pl.ds(base, K), pl.ds(p * 32, 32)],
                             wij_v.at[:, pl.ds(0, 32)], sem)
            pltpu.async_copy(wij_hbm.at[pl.ds(base, K), pl.ds(128 + p * 32, 32)],
                             wij_v.at[:, pl.ds(32, 32)], sem)
            pltpu.async_copy(wij_hbm.at[pl.ds(base, K), pl.ds(256 + p * 32, 32)],
                             wij_v.at[:, pl.ds(64, 32)], sem)
            pltpu.async_copy(dir_hbm.at[pl.ds(base, K)], dir_v, sem)

        def scat_wait(slot):
            _, _, _, _, idxi_v, ctb_v, _, ssem = slot
            pltpu.make_async_copy(ctb_v, acc.at[idxi_v], ssem).wait()

        def process(t, slot, has_prev):
            rows_v, wij_v, dir_v, idxj_v, idxi_v, ctb_v, sem, ssem = slot

            @pl.when(has_prev)
            def _():
                scat_wait(slot)

            pltpu.async_copy(idxi_hbm.at[pl.ds(t * K, K)], idxi_v, sem)
            pltpu.make_async_copy(table_hbm.at[idxj_v], rows_v, sem).wait()
            pltpu.make_async_copy(wij_hbm.at[pl.ds(0, K), pl.ds(0, 96)],
                                  wij_v, sem).wait()
            pltpu.make_async_copy(dir_hbm.at[pl.ds(0, K)], dir_v, sem).wait()

            def edge_body(e, _):
                sv = rows_v[e, pl.ds(96, 16)]
                s0 = sv[0]
                s1 = sv[1]
                s2 = sv[2]
                dv = dir_v[e, pl.ds(0, 16)]
                d0 = dv[0]
                d1 = dv[1]
                d2 = dv[2]
                for h in (0, 16):
                    x1 = rows_v[e, pl.ds(h, 16)]
                    x2 = rows_v[e, pl.ds(32 + h, 16)]
                    x3 = rows_v[e, pl.ds(64 + h, 16)]
                    wq = wij_v[e, pl.ds(h, 16)]
                    wr = wij_v[e, pl.ds(32 + h, 16)]
                    wm = wij_v[e, pl.ds(64 + h, 16)]
                    av = wr * x2
                    bv = wm * x3
                    ctb_v[e, pl.ds(h, 16)] = wq * x1
                    ctb_v[e, pl.ds(32 + h, 16)] = av * d0 + bv * s0
                    ctb_v[e, pl.ds(64 + h, 16)] = av * d1 + bv * s1
                    ctb_v[e, pl.ds(96 + h, 16)] = av * d2 + bv * s2
                return 0

            lax.fori_loop(0, K, edge_body, 0)
            pltpu.make_async_copy(idxi_hbm.at[pl.ds(0, K)], idxi_v, sem).wait()
            pltpu.async_copy(ctb_v, acc.at[idxi_v], ssem, add=True)

        for p in range(4):
            pltpu.sync_copy(zeros_hbm, acc.at[pl.ds(sid * rpt, rpt)])
            plsc.subcore_barrier()
            issue(p, wid, slots[0])

            def pair_body(i, _, p=p):
                t_a = wid + (2 * i) * NW
                t_b = wid + (2 * i + 1) * NW
                t_a2 = wid + (2 * i + 2) * NW

                @pl.when(t_b < NB)
                def _():
                    issue(p, t_b, slots[1])

                @pl.when(t_a < NB)
                def _():
                    process(t_a, slots[0], i > 0)

                @pl.when(t_a2 < NB)
                def _():
                    issue(p, t_a2, slots[0])

                @pl.when(t_b < NB)
                def _():
                    process(t_b, slots[1], i > 0)

                return 0

            lax.fori_loop(0, total_k // 2, pair_body, 0)
            scat_wait(slots[0])
            scat_wait(slots[1])
            plsc.subcore_barrier()
            pltpu.sync_copy(acc.at[pl.ds(sid * rpt, rpt)],
                            out_hbm.at[cid, p, pl.ds(sid * rpt, rpt)])
            plsc.subcore_barrier()

    return sc_fn(table, idxj4, idx_i, wij, dir_ij, zeros)


def _stage3_body(q_ref, dq_ref, dmu_ref, w_ref, s_ref,
                 Wr1_ref, br1_ref, Wr2T_ref, br2T_ref,
                 qn_ref, mun_ref, dtm_ref):
    B = q_ref.shape[0]
    w = w_ref[...]
    s = s_ref[...]
    mu_new = dmu_ref[...] + w[:, None, :] * s[:, :, None]
    mun_ref[...] = mu_new
    qn_ref[...] = q_ref[...] + dq_ref[...][:, None, :]
    inv2 = jnp.sqrt(mu_new[:, 0, :] ** 2 + mu_new[:, 1, :] ** 2
                    + mu_new[:, 2, :] ** 2)
    h2 = jnp.maximum(jnp.dot(inv2, Wr1_ref[...], preferred_element_type=jnp.float32)
                     + br1_ref[...], 0.0)
    sw2t = jnp.dot(h2, Wr2T_ref[...], preferred_element_type=jnp.float32) \
        + br2T_ref[...]
    zrows = jnp.zeros((7, F), jnp.float32)
    for b in range(B):
        mb0 = mu_new[b:b + 1, 0, :]
        mb1 = mu_new[b:b + 1, 1, :]
        mb2 = mu_new[b:b + 1, 2, :]
        rows = [mb0 * mb0, mb0 * mb1, mb0 * mb2,
                mb1 * mb0, mb1 * mb1, mb1 * mb2,
                mb2 * mb0, mb2 * mb1, mb2 * mb2]
        rt16 = jnp.concatenate(rows + [zrows], axis=0)
        swbt = sw2t[b:b + 1, :].reshape(F, F)
        dt16 = jnp.dot(rt16, swbt, preferred_element_type=jnp.float32)
        dtm_ref[b, :, :] = dt16[0:9, :]


def _stage3(q, dq, dmu, w, s, Wr1, br1, Wr2T, br2T):
    N = q.shape[0]
    B = 8
    grid = N // B
    full = lambda *shape: pl.BlockSpec(shape, lambda i: (0,) * len(shape))
    return pl.pallas_call(
        functools.partial(_stage3_body),
        grid=(grid,),
        in_specs=[
            pl.BlockSpec((B, 1, F), lambda i: (i, 0, 0)),
            pl.BlockSpec((B, F), lambda i: (i, 0)),
            pl.BlockSpec((B, 3, F), lambda i: (i, 0, 0)),
            pl.BlockSpec((B, F), lambda i: (i, 0)),
            pl.BlockSpec((B, 3), lambda i: (i, 0)),
            full(F, 32), full(1, 32), full(32, F * F), full(1, F * F),
        ],
        out_specs=[
            pl.BlockSpec((B, 1, F), lambda i: (i, 0, 0)),
            pl.BlockSpec((B, 3, F), lambda i: (i, 0, 0)),
            pl.BlockSpec((B, 9, F), lambda i: (i, 0, 0)),
        ],
        out_shape=[
            jax.ShapeDtypeStruct((N, 1, F), jnp.float32),
            jax.ShapeDtypeStruct((N, 3, F), jnp.float32),
            jax.ShapeDtypeStruct((N, 9, F), jnp.float32),
        ],
    )(q, dq, dmu, w, s, Wr1, br1.reshape(1, -1), Wr2T, br2T.reshape(1, -1))


def kernel(q, mu, Wij, dir_ij, idx_i, idx_j, n_atoms,
           W1, b1, W2, b2, Wc1, bc1, Wc2, bc2, Wr1, br1, Wr2, br2):
    N = q.shape[0]
    E = idx_i.shape[0]

    x, w, s = _stage1(q, mu, W1, b1, W2, b2, Wc1, bc1, Wc2, bc2)

    xr = x.reshape(N, 3, 4, 32).transpose(2, 0, 1, 3).reshape(4, N, 96)
    sr = jnp.broadcast_to(s[None], (4, N, 3))
    pad = jnp.zeros((4, N, 13), jnp.float32)
    table = jnp.concatenate([xr, sr, pad], axis=-1).reshape(4 * N, 112)
    idxj4 = (idx_j[None, :] + (jnp.arange(4, dtype=jnp.int32) * N)[:, None])
    idxj4 = idxj4.reshape(-1)
    zeros = jnp.zeros((N // 16, F), jnp.float32)
    dir16 = jnp.concatenate(
        [dir_ij, jnp.zeros((E, 13), jnp.float32)], axis=-1)

    partials = _sc_edges(table, idxj4, idx_i, Wij.reshape(E, 3 * F),
                         dir16, zeros)

    P = partials[0] + partials[1]
    Pr = P.reshape(4, N, 4, 32)
    dq = Pr[:, :, 0, :].transpose(1, 0, 2).reshape(N, F)
    dmu = Pr[:, :, 1:4, :].transpose(1, 2, 0, 3).reshape(N, 3, F)

    Wr2T = Wr2.reshape(32, F, F).transpose(0, 2, 1).reshape(32, F * F)
    br2T = br2.reshape(F, F).T.reshape(F * F)
    q_new, mu_new, dtm9 = _stage3(q, dq, dmu, w, s, Wr1, br1, Wr2T, br2T)
    dtm = dtm9.transpose(0, 2, 1).reshape(N, F, 3, 3)
    return (q_new, mu_new, dtm)


# speedup vs baseline: 1.5792x; 1.0703x over previous
import functools

import jax
import jax.numpy as jnp
from jax import lax
from jax.experimental import pallas as pl
from jax.experimental.pallas import tpu as pltpu
from jax.experimental.pallas import tpu_sc as plsc

F = 128


def _stage1_body(q_ref, mu_ref, W1_ref, b1_ref, W2_ref, b2_ref,
                 Wc1_ref, bc1_ref, Wc2_ref, bc2_ref,
                 x_ref, w_ref, s_ref):
    q2 = q_ref[:, 0, :]
    a = jnp.dot(q2, W1_ref[...], preferred_element_type=jnp.float32) + b1_ref[...]
    a = a * jax.nn.sigmoid(a)
    x = jnp.dot(a, W2_ref[...], preferred_element_type=jnp.float32) + b2_ref[...]
    mu = mu_ref[...]
    inv = jnp.sqrt(mu[:, 0, :] ** 2 + mu[:, 1, :] ** 2 + mu[:, 2, :] ** 2)
    h = jnp.maximum(jnp.dot(inv, Wc1_ref[...], preferred_element_type=jnp.float32)
                    + bc1_ref[...], 0.0)
    logit = jnp.dot(h, Wc2_ref[...], preferred_element_type=jnp.float32) + bc2_ref[...]
    m = jnp.max(logit, axis=-1, keepdims=True)
    e = jnp.exp(logit - m)
    w = e / jnp.sum(e, axis=-1, keepdims=True)
    w_ref[...] = w
    x_ref[...] = jnp.concatenate(
        [x[:, 0:256], x[:, 256:384] * w], axis=-1)
    s_ref[...] = jnp.sum(mu, axis=2)


def _stage1(q, mu, W1, b1, W2, b2, Wc1, bc1, Wc2, bc2):
    N = q.shape[0]
    B = 400
    grid = N // B
    full = lambda *shape: pl.BlockSpec(shape, lambda i: (0,) * len(shape))
    return pl.pallas_call(
        _stage1_body,
        grid=(grid,),
        in_specs=[
            pl.BlockSpec((B, 1, F), lambda i: (i, 0, 0)),
            pl.BlockSpec((B, 3, F), lambda i: (i, 0, 0)),
            full(F, F), full(1, F), full(F, 3 * F), full(1, 3 * F),
            full(F, 32), full(1, 32), full(32, F), full(1, F),
        ],
        out_specs=[
            pl.BlockSpec((B, 3 * F), lambda i: (i, 0)),
            pl.BlockSpec((B, F), lambda i: (i, 0)),
            pl.BlockSpec((B, 3), lambda i: (i, 0)),
        ],
        out_shape=[
            jax.ShapeDtypeStruct((N, 3 * F), jnp.float32),
            jax.ShapeDtypeStruct((N, F), jnp.float32),
            jax.ShapeDtypeStruct((N, 3), jnp.float32),
        ],
    )(q, mu, W1, b1.reshape(1, -1), W2, b2.reshape(1, -1),
      Wc1, bc1.reshape(1, -1), Wc2, bc2.reshape(1, -1))


def _sc_edges(table, idxj4, idx_i, wij, dir_ij, zeros):
    E = idx_i.shape[0]
    N = table.shape[0] // 4
    K = 64
    NB = E // K
    assert NB * K == E and K % 8 == 0
    NW = 32
    total_k = (NB + NW - 1) // NW
    total_k += total_k % 2
    rpt = N // 16
    mesh = plsc.VectorSubcoreMesh(core_axis_name="c", subcore_axis_name="s")

    slot_t = [
        pltpu.VMEM((K, 112), jnp.float32),
        pltpu.VMEM((K, 96), jnp.float32),
        pltpu.VMEM((K, 16), jnp.float32),
        pltpu.VMEM((K,), jnp.int32),
        pltpu.VMEM((K,), jnp.int32),
        pltpu.VMEM((K, F), jnp.float32),
        pltpu.SemaphoreType.DMA,
        pltpu.SemaphoreType.DMA,
        pltpu.SemaphoreType.DMA,
        pltpu.SemaphoreType.DMA,
    ]

    @functools.partial(
        pl.kernel, mesh=mesh,
        compiler_params=pltpu.CompilerParams(use_tc_tiling_on_sc=False),
        out_type=jax.ShapeDtypeStruct((2, 4, N, F), jnp.float32),
        scratch_types=slot_t + slot_t + [
            pltpu.VMEM_SHARED((N, F), jnp.float32),
        ],
    )
    def sc_fn(table_hbm, idxj_hbm, idxi_hbm, wij_hbm, dir_hbm, zeros_hbm,
              out_hbm,
              rows_a, wij_a, dir_a, idxj_a, idxi_a, ctb_a,
              sem_a, ssem_a, jsem_a, gsem_a,
              rows_b, wij_b, dir_b, idxj_b, idxi_b, ctb_b,
              sem_b, ssem_b, jsem_b, gsem_b,
              acc):
        cid = lax.axis_index("c")
        sid = lax.axis_index("s")
        wid = sid * 2 + cid
        slots = (
            (rows_a, wij_a, dir_a, idxj_a, idxi_a, ctb_a,
             sem_a, ssem_a, jsem_a, gsem_a),
            (rows_b, wij_b, dir_b, idxj_b, idxi_b, ctb_b,
             sem_b, ssem_b, jsem_b, gsem_b))

        def issue_gathers(p, t, slot):
            (rows_v, wij_v, dir_v, idxj_v, idxi_v, ctb_v,
             sem, ssem, jsem, gsem) = slot
            base = t * K
            pltpu.async_copy(table_hbm.at[idxj_v], rows_v, gsem)
            pltpu.async_copy(wij_hbm.at[pl.ds(base, K), pl.ds(p * 32, 32)],
                             wij_v.at[:, pl.ds(0, 32)], sem)
            pltpu.async_copy(wij_hbm.at[pl.ds(base, K), pl.ds(128 + p * 32, 32)],
                             wij_v.at[:, pl.ds(32, 32)], sem)
            pltpu.async_copy(wij_hbm.at[pl.ds(base, K), pl.ds(256 + p * 32, 32)],
                             wij_v.at[:, pl.ds(64, 32)], sem)
            pltpu.async_copy(dir_hbm.at[pl.ds(base, K)], dir_v, sem)

        def issue(p, t, slot):
            idxj_v, jsem = slot[3], slot[8]
            pltpu.make_async_copy(idxj_hbm.at[pl.ds(0, K)], idxj_v, jsem).wait()
            issue_gathers(p, t, slot)

        def scat_wait(slot):
            idxi_v, ctb_v, ssem = slot[4], slot[5], slot[7]
            pltpu.make_async_copy(ctb_v, acc.at[idxi_v], ssem).wait()

        def process(p, t, slot, has_prev):
            (rows_v, wij_v, dir_v, idxj_v, idxi_v, ctb_v,
             sem, ssem, jsem, gsem) = slot

            @pl.when(has_prev)
            def _():
                scat_wait(slot)

            pltpu.async_copy(idxi_hbm.at[pl.ds(t * K, K)], idxi_v, ssem)
            pltpu.make_async_copy(table_hbm.at[idxj_v], rows_v, gsem).wait()
            pltpu.make_async_copy(wij_hbm.at[pl.ds(0, K), pl.ds(0, 96)],
                                  wij_v, sem).wait()
            pltpu.make_async_copy(dir_hbm.at[pl.ds(0, K)], dir_v, sem).wait()
            t2 = t + 2 * NW

            @pl.when(t2 < NB)
            def _():
                pltpu.async_copy(idxj_hbm.at[pl.ds(p * E + t2 * K, K)],
                                 idxj_v, jsem)

            def edge_body(e, _):
                sv = rows_v[e, pl.ds(96, 16)]
                s0 = sv[0]
                s1 = sv[1]
                s2 = sv[2]
                dv = dir_v[e, pl.ds(0, 16)]
                d0 = dv[0]
                d1 = dv[1]
                d2 = dv[2]
                for h in (0, 16):
                    x1 = rows_v[e, pl.ds(h, 16)]
                    x2 = rows_v[e, pl.ds(32 + h, 16)]
                    x3 = rows_v[e, pl.ds(64 + h, 16)]
                    wq = wij_v[e, pl.ds(h, 16)]
                    wr = wij_v[e, pl.ds(32 + h, 16)]
                    wm = wij_v[e, pl.ds(64 + h, 16)]
                    av = wr * x2
                    bv = wm * x3
                    ctb_v[e, pl.ds(h, 16)] = wq * x1
                    ctb_v[e, pl.ds(32 + h, 16)] = av * d0 + bv * s0
                    ctb_v[e, pl.ds(64 + h, 16)] = av * d1 + bv * s1
                    ctb_v[e, pl.ds(96 + h, 16)] = av * d2 + bv * s2
                return 0

            lax.fori_loop(0, K, edge_body, 0)
            pltpu.make_async_copy(idxi_hbm.at[pl.ds(0, K)], idxi_v, ssem).wait()
            pltpu.async_copy(ctb_v, acc.at[idxi_v], ssem, add=True)

        for p in range(4):
            pltpu.sync_copy(zeros_hbm, acc.at[pl.ds(sid * rpt, rpt)])
            plsc.subcore_barrier()
            pltpu.sync_copy(idxj_hbm.at[pl.ds(p * E + wid * K, K)],
                            slots[0][3])
            issue_gathers(p, wid, slots[0])
            pltpu.async_copy(idxj_hbm.at[pl.ds(p * E + (wid + NW) * K, K)],
                             slots[1][3], slots[1][8])

            def pair_body(i, _, p=p):
                t_a = wid + (2 * i) * NW
                t_b = wid + (2 * i + 1) * NW
                t_a2 = wid + (2 * i + 2) * NW

                @pl.when(t_b < NB)
                def _():
                    issue(p, t_b, slots[1])

                @pl.when(t_a < NB)
                def _():
                    process(p, t_a, slots[0], i > 0)

                @pl.when(t_a2 < NB)
                def _():
                    issue(p, t_a2, slots[0])

                @pl.when(t_b < NB)
                def _():
                    process(p, t_b, slots[1], i > 0)

                return 0

            lax.fori_loop(0, total_k // 2, pair_body, 0)
            scat_wait(slots[0])
            scat_wait(slots[1])
            plsc.subcore_barrier()
            pltpu.sync_copy(acc.at[pl.ds(sid * rpt, rpt)],
                            out_hbm.at[cid, p, pl.ds(sid * rpt, rpt)])
            plsc.subcore_barrier()

    return sc_fn(table, idxj4, idx_i, wij, dir_ij, zeros)


def _stage3_body(q_ref, dq_ref, dmu_ref, w_ref, s_ref,
                 Wr1_ref, br1_ref, Wr2T_ref, br2T_ref,
                 qn_ref, mun_ref, dtm_ref):
    B = q_ref.shape[0]
    w = w_ref[...]
    s = s_ref[...]
    mu_new = dmu_ref[...] + w[:, None, :] * s[:, :, None]
    mun_ref[...] = mu_new
    qn_ref[...] = q_ref[...] + dq_ref[...][:, None, :]
    inv2 = jnp.sqrt(mu_new[:, 0, :] ** 2 + mu_new[:, 1, :] ** 2
                    + mu_new[:, 2, :] ** 2)
    h2 = jnp.maximum(jnp.dot(inv2, Wr1_ref[...], preferred_element_type=jnp.float32)
                     + br1_ref[...], 0.0)
    sw2t = jnp.dot(h2, Wr2T_ref[...], preferred_element_type=jnp.float32) \
        + br2T_ref[...]
    zrows = jnp.zeros((7, F), jnp.float32)
    for b in range(B):
        mb0 = mu_new[b:b + 1, 0, :]
        mb1 = mu_new[b:b + 1, 1, :]
        mb2 = mu_new[b:b + 1, 2, :]
        rows = [mb0 * mb0, mb0 * mb1, mb0 * mb2,
                mb1 * mb0, mb1 * mb1, mb1 * mb2,
                mb2 * mb0, mb2 * mb1, mb2 * mb2]
        rt16 = jnp.concatenate(rows + [zrows], axis=0)
        swbt = sw2t[b:b + 1, :].reshape(F, F)
        dt16 = jnp.dot(rt16, swbt, preferred_element_type=jnp.float32)
        dtm_ref[b, :, :] = dt16[0:9, :]


def _stage3(q, dq, dmu, w, s, Wr1, br1, Wr2T, br2T):
    N = q.shape[0]
    B = 8
    grid = N // B
    full = lambda *shape: pl.BlockSpec(shape, lambda i: (0,) * len(shape))
    return pl.pallas_call(
        functools.partial(_stage3_body),
        grid=(grid,),
        in_specs=[
            pl.BlockSpec((B, 1, F), lambda i: (i, 0, 0)),
            pl.BlockSpec((B, F), lambda i: (i, 0)),
            pl.BlockSpec((B, 3, F), lambda i: (i, 0, 0)),
            pl.BlockSpec((B, F), lambda i: (i, 0)),
            pl.BlockSpec((B, 3), lambda i: (i, 0)),
            full(F, 32), full(1, 32), full(32, F * F), full(1, F * F),
        ],
        out_specs=[
            pl.BlockSpec((B, 1, F), lambda i: (i, 0, 0)),
            pl.BlockSpec((B, 3, F), lambda i: (i, 0, 0)),
            pl.BlockSpec((B, 9, F), lambda i: (i, 0, 0)),
        ],
        out_shape=[
            jax.ShapeDtypeStruct((N, 1, F), jnp.float32),
            jax.ShapeDtypeStruct((N, 3, F), jnp.float32),
            jax.ShapeDtypeStruct((N, 9, F), jnp.float32),
        ],
    )(q, dq, dmu, w, s, Wr1, br1.reshape(1, -1), Wr2T, br2T.reshape(1, -1))


def kernel(q, mu, Wij, dir_ij, idx_i, idx_j, n_atoms,
           W1, b1, W2, b2, Wc1, bc1, Wc2, bc2, Wr1, br1, Wr2, br2):
    N = q.shape[0]
    E = idx_i.shape[0]

    x, w, s = _stage1(q, mu, W1, b1, W2, b2, Wc1, bc1, Wc2, bc2)

    xr = x.reshape(N, 3, 4, 32).transpose(2, 0, 1, 3).reshape(4, N, 96)
    sr = jnp.broadcast_to(s[None], (4, N, 3))
    pad = jnp.zeros((4, N, 13), jnp.float32)
    table = jnp.concatenate([xr, sr, pad], axis=-1).reshape(4 * N, 112)
    idxj4 = (idx_j[None, :] + (jnp.arange(4, dtype=jnp.int32) * N)[:, None])
    idxj4 = idxj4.reshape(-1)
    zeros = jnp.zeros((N // 16, F), jnp.float32)
    dir16 = jnp.concatenate(
        [dir_ij, jnp.zeros((E, 13), jnp.float32)], axis=-1)

    partials = _sc_edges(table, idxj4, idx_i, Wij.reshape(E, 3 * F),
                         dir16, zeros)

    P = partials[0] + partials[1]
    Pr = P.reshape(4, N, 4, 32)
    dq = Pr[:, :, 0, :].transpose(1, 0, 2).reshape(N, F)
    dmu = Pr[:, :, 1:4, :].transpose(1, 2, 0, 3).reshape(N, 3, F)

    Wr2T = Wr2.reshape(32, F, F).transpose(0, 2, 1).reshape(32, F * F)
    br2T = br2.reshape(F, F).T.reshape(F * F)
    q_new, mu_new, dtm9 = _stage3(q, dq, dmu, w, s, Wr1, br1, Wr2T, br2T)
    dtm = dtm9.transpose(0, 2, 1).reshape(N, F, 3, 3)
    return (q_new, mu_new, dtm)
